# baseline (speedup 1.0000x reference)
import jax
import jax.numpy as jnp
import numpy as np
from jax.experimental import pallas as pl
from jax.experimental.pallas import tpu as pltpu

N_RBF = 16
N_HIDDEN = 32
CUTOFF = 5.0
N_ATOMS = 2048
BLOCK = 256
NB = N_ATOMS // BLOCK
NPAIR = NB * (NB + 1) // 2


def _rbf_consts():
    c = np.linspace(0.0, CUTOFF, N_RBF, dtype=np.float32).astype(np.float64)
    width32 = np.float32(np.float32(c[1]) - np.float32(c[0]))
    coeff = np.float64(np.float32(-0.5 / (width32 * width32)))
    w = (c[N_RBF - 1] - c[0]) / (N_RBF - 1)
    a = -2.0 * coeff * w
    bases = [c[0], c[8]]
    scales = np.array(
        [np.exp(coeff * (c[k] - bases[k // 8]) ** 2) for k in range(N_RBF)],
        dtype=np.float64,
    )
    return (np.float32(coeff), np.float32(a),
            [np.float32(b) for b in bases], scales.astype(np.float32))


def _body(idx_ref, rowpos_ref, posT_ref, W1T_ref, b1_ref, W2T_ref, b2_ref,
          W3T_ref, b3_ref, out_ref, feat_ref):
    coeff, a, bases, scales = _rbf_consts()
    t_id = pl.program_id(0)
    bi = idx_ref[0, t_id]
    bj = idx_ref[1, t_id]
    diag = bi == bj

    @pl.when(t_id == 0)
    def _init():
        feat_ref[...] = jnp.zeros((N_RBF, N_ATOMS), jnp.float32)

    rowpos = rowpos_ref[pl.ds(bi * BLOCK, BLOCK), :]
    posT = posT_ref[:, pl.ds(bj * BLOCK, BLOCK)]
    dx = rowpos[:, 0:1] - posT[0:1, :]
    dy = rowpos[:, 1:2] - posT[1:2, :]
    dz = rowpos[:, 2:3] - posT[2:3, :]
    d2 = dx * dx + dy * dy + dz * dz
    dist = jnp.sqrt(d2)

    beyond = dist >= CUTOFF
    u0 = jnp.minimum(dist, 29.0)
    s0 = jnp.exp(coeff * (u0 * u0))
    tt0 = jnp.exp(a * u0)
    u1 = jnp.where(beyond, 100.0, dist - bases[1])
    s1 = jnp.exp(coeff * (u1 * u1))
    tt1 = tt0 * float(np.exp(np.float64(-a) * np.float64(bases[1])))
    rows = []
    cols = []
    for g, (s, tt) in enumerate(((s0, tt0), (s1, tt1))):
        p = s
        for m in range(8):
            if m:
                p = p * tt
            sc = float(scales[8 * g + m])
            rows.append(jnp.sum(p, axis=1) * sc)
            cols.append(jnp.sum(p, axis=0) * sc)
    row_tile = jnp.stack(rows, axis=0)
    col_tile = jnp.stack(cols, axis=0)

    sl_i = pl.ds(bi * BLOCK, BLOCK)
    feat_ref[:, sl_i] = feat_ref[:, sl_i] + row_tile

    @pl.when(jnp.logical_not(diag))
    def _offdiag():
        sl_j = pl.ds(bj * BLOCK, BLOCK)
        feat_ref[:, sl_j] = feat_ref[:, sl_j] + col_tile

    @pl.when(t_id == NPAIR - 1)
    def _mlp():
        feat = feat_ref[...]
        h = jax.lax.dot_general(W1T_ref[...], feat, (((1,), (0,)), ((), ())),
                                preferred_element_type=jnp.float32)
        h = jax.nn.silu(h + b1_ref[...])
        h = jax.lax.dot_general(W2T_ref[...], h, (((1,), (0,)), ((), ())),
                                preferred_element_type=jnp.float32)
        h = jax.nn.silu(h + b2_ref[...])
        e = jax.lax.dot_general(W3T_ref[...], h, (((1,), (0,)), ((), ())),
                                preferred_element_type=jnp.float32)
        energy = jnp.sum(e + b3_ref[...])
        out_ref[...] = jnp.broadcast_to(energy, (1, 128))


def kernel(positions, W1, b1, W2, b2, W3, b3):
    f32 = jnp.float32
    positions = positions.astype(f32)
    rowpos = jnp.zeros((N_ATOMS, 8), f32).at[:, :3].set(positions)
    posT = jnp.zeros((8, N_ATOMS), f32).at[:3, :].set(positions.T)

    W1T = W1.T.astype(f32)
    W2T = W2.T.astype(f32)
    W3T = W3.T.astype(f32)
    c64 = np.linspace(0.0, CUTOFF, N_RBF, dtype=np.float32).astype(np.float64)
    w32 = np.float32(np.float32(c64[1]) - np.float32(c64[0]))
    coeff64 = np.float64(np.float32(-0.5 / (w32 * w32)))
    selfvec = np.exp(coeff64 * c64 * c64).astype(np.float32)[:, None]
    b1c = b1.astype(f32)[:, None] - W1T @ jnp.asarray(selfvec)
    b2c = b2.astype(f32)[:, None]
    b3c = b3.astype(f32)[:, None]

    pair_idx = np.array(
        [[i for i in range(NB) for _ in range(i, NB)],
         [j for i in range(NB) for j in range(i, NB)]], dtype=np.int32)

    grid_spec = pltpu.PrefetchScalarGridSpec(
        num_scalar_prefetch=1,
        grid=(NPAIR,),
        in_specs=[
            pl.BlockSpec((N_ATOMS, 8), lambda t, idx: (0, 0)),
            pl.BlockSpec((8, N_ATOMS), lambda t, idx: (0, 0)),
            pl.BlockSpec(W1T.shape, lambda t, idx: (0, 0)),
            pl.BlockSpec(b1c.shape, lambda t, idx: (0, 0)),
            pl.BlockSpec(W2T.shape, lambda t, idx: (0, 0)),
            pl.BlockSpec(b2c.shape, lambda t, idx: (0, 0)),
            pl.BlockSpec(W3T.shape, lambda t, idx: (0, 0)),
            pl.BlockSpec(b3c.shape, lambda t, idx: (0, 0)),
        ],
        out_specs=pl.BlockSpec((1, 128), lambda t, idx: (0, 0)),
        scratch_shapes=[pltpu.VMEM((N_RBF, N_ATOMS), f32)],
    )
    out = pl.pallas_call(
        _body,
        grid_spec=grid_spec,
        out_shape=jax.ShapeDtypeStruct((1, 128), f32),
        compiler_params=pltpu.CompilerParams(
            dimension_semantics=("arbitrary",),
        ),
    )(jnp.asarray(pair_idx), rowpos, posT, W1T, b1c, W2T, b2c, W3T, b3c)
    return out[0, 0]


# speedup vs baseline: 1.0070x; 1.0070x over previous
import jax
import jax.numpy as jnp
import numpy as np
from jax.experimental import pallas as pl
from jax.experimental.pallas import tpu as pltpu

N_RBF = 16
N_HIDDEN = 32
CUTOFF = 5.0
N_ATOMS = 2048
BLOCK = 256
NB = N_ATOMS // BLOCK
NPAIR = NB * (NB + 1) // 2


def _rbf_consts():
    c = np.linspace(0.0, CUTOFF, N_RBF, dtype=np.float32).astype(np.float64)
    width32 = np.float32(np.float32(c[1]) - np.float32(c[0]))
    coeff = np.float64(np.float32(-0.5 / (width32 * width32)))
    w = (c[N_RBF - 1] - c[0]) / (N_RBF - 1)
    a = -2.0 * coeff * w
    bases = [c[0], c[8]]
    scales = np.array(
        [np.exp(coeff * (c[k] - bases[k // 8]) ** 2) for k in range(N_RBF)],
        dtype=np.float64,
    )
    return (np.float32(coeff), np.float32(a),
            [np.float32(b) for b in bases], scales.astype(np.float32))


def _body(idx_ref, rowpos_ref, posT_ref, W1T_ref, b1_ref, W2T_ref, b2_ref,
          W3T_ref, b3_ref, out_ref, feat_ref):
    coeff, a, bases, scales = _rbf_consts()
    t_id = pl.program_id(0)
    bi = idx_ref[0, t_id]
    bj = idx_ref[1, t_id]
    diag = bi == bj

    @pl.when(t_id == 0)
    def _init():
        feat_ref[...] = jnp.zeros((N_RBF, N_ATOMS), jnp.float32)

    rowpos = rowpos_ref[pl.ds(bi * BLOCK, BLOCK), :]
    posT = posT_ref[:, pl.ds(bj * BLOCK, BLOCK)]
    dx = rowpos[:, 0:1] - posT[0:1, :]
    dy = rowpos[:, 1:2] - posT[1:2, :]
    dz = rowpos[:, 2:3] - posT[2:3, :]
    d2 = dx * dx + dy * dy + dz * dz
    dist = jnp.sqrt(d2)

    beyond = dist >= CUTOFF
    u0 = jnp.minimum(dist, 29.0)
    s0 = jnp.exp(coeff * (u0 * u0))
    tt0 = jnp.exp(a * u0)
    u1 = jnp.where(beyond, 100.0, dist - bases[1])
    s1 = jnp.exp(coeff * (u1 * u1))
    tt1 = tt0 * float(np.exp(np.float64(-a) * np.float64(bases[1])))
    rows = [None] * N_RBF
    cols = [None] * N_RBF
    p0, p1 = s0, s1
    for m in range(8):
        if m:
            p0 = p0 * tt0
            p1 = p1 * tt1
        for g, p in ((0, p0), (1, p1)):
            k = 8 * g + m
            sc = float(scales[k])
            rows[k] = jnp.sum(p, axis=1) * sc
            cols[k] = jnp.sum(p, axis=0) * sc
    row_tile = jnp.stack(rows, axis=0)
    col_tile = jnp.stack(cols, axis=0)

    sl_i = pl.ds(bi * BLOCK, BLOCK)
    feat_ref[:, sl_i] = feat_ref[:, sl_i] + row_tile

    @pl.when(jnp.logical_not(diag))
    def _offdiag():
        sl_j = pl.ds(bj * BLOCK, BLOCK)
        feat_ref[:, sl_j] = feat_ref[:, sl_j] + col_tile

    @pl.when(t_id == NPAIR - 1)
    def _mlp():
        feat = feat_ref[...]
        h = jax.lax.dot_general(W1T_ref[...], feat, (((1,), (0,)), ((), ())),
                                preferred_element_type=jnp.float32)
        h = jax.nn.silu(h + b1_ref[...])
        h = jax.lax.dot_general(W2T_ref[...], h, (((1,), (0,)), ((), ())),
                                preferred_element_type=jnp.float32)
        h = jax.nn.silu(h + b2_ref[...])
        e = jax.lax.dot_general(W3T_ref[...], h, (((1,), (0,)), ((), ())),
                                preferred_element_type=jnp.float32)
        energy = jnp.sum(e + b3_ref[...])
        out_ref[...] = jnp.broadcast_to(energy, (1, 128))


def kernel(positions, W1, b1, W2, b2, W3, b3):
    f32 = jnp.float32
    positions = positions.astype(f32)
    rowpos = jnp.zeros((N_ATOMS, 8), f32).at[:, :3].set(positions)
    posT = jnp.zeros((8, N_ATOMS), f32).at[:3, :].set(positions.T)

    W1T = W1.T.astype(f32)
    W2T = W2.T.astype(f32)
    W3T = W3.T.astype(f32)
    c64 = np.linspace(0.0, CUTOFF, N_RBF, dtype=np.float32).astype(np.float64)
    w32 = np.float32(np.float32(c64[1]) - np.float32(c64[0]))
    coeff64 = np.float64(np.float32(-0.5 / (w32 * w32)))
    selfvec = np.exp(coeff64 * c64 * c64).astype(np.float32)[:, None]
    b1c = b1.astype(f32)[:, None] - W1T @ jnp.asarray(selfvec)
    b2c = b2.astype(f32)[:, None]
    b3c = b3.astype(f32)[:, None]

    pair_idx = np.array(
        [[i for i in range(NB) for _ in range(i, NB)],
         [j for i in range(NB) for j in range(i, NB)]], dtype=np.int32)

    grid_spec = pltpu.PrefetchScalarGridSpec(
        num_scalar_prefetch=1,
        grid=(NPAIR,),
        in_specs=[
            pl.BlockSpec((N_ATOMS, 8), lambda t, idx: (0, 0)),
            pl.BlockSpec((8, N_ATOMS), lambda t, idx: (0, 0)),
            pl.BlockSpec(W1T.shape, lambda t, idx: (0, 0)),
            pl.BlockSpec(b1c.shape, lambda t, idx: (0, 0)),
            pl.BlockSpec(W2T.shape, lambda t, idx: (0, 0)),
            pl.BlockSpec(b2c.shape, lambda t, idx: (0, 0)),
            pl.BlockSpec(W3T.shape, lambda t, idx: (0, 0)),
            pl.BlockSpec(b3c.shape, lambda t, idx: (0, 0)),
        ],
        out_specs=pl.BlockSpec((1, 128), lambda t, idx: (0, 0)),
        scratch_shapes=[pltpu.VMEM((N_RBF, N_ATOMS), f32)],
    )
    out = pl.pallas_call(
        _body,
        grid_spec=grid_spec,
        out_shape=jax.ShapeDtypeStruct((1, 128), f32),
        compiler_params=pltpu.CompilerParams(
            dimension_semantics=("arbitrary",),
        ),
    )(jnp.asarray(pair_idx), rowpos, posT, W1T, b1c, W2T, b2c, W3T, b3c)
    return out[0, 0]
